# baseline (device time: 48336 ns/iter reference)
import functools

import jax
import jax.numpy as jnp
from jax import lax
from jax.experimental import pallas as pl
from jax.experimental.pallas import tpu as pltpu

N_DEV = 4
NEG_INF = -1e9
BAND = 128
NGLOB = 32


def kernel(x, Wq, K_ext, V_ext, Wo):
    B, Sq, Dm = x.shape
    _, Skv, Hq, Dh = K_ext.shape
    Dqk = Hq * Dh
    NCOL = Skv + 2 * BAND + NGLOB
    NROW = Sq + NGLOB

    x2 = x.reshape(B * Sq, Dm)
    k2 = K_ext.reshape(B * Skv, Dqk)
    v2 = V_ext.reshape(B * Skv, Dqk)

    def body(x_ref, wq_ref, k_ref, v_ref, wo_ref, out_ref,
             ho_r, ho_l, hi_l, hi_r, gk_out, gk_in, qg_out, qg_in,
             part_out, part_in,
             hsend, hrecv, qgsend, qgrecv, gksend, gkrecv, psend, precv):
        my = lax.axis_index("i")
        left = lax.rem(my + N_DEV - 1, N_DEV)
        right = lax.rem(my + 1, N_DEV)
        is_root = my == 0

        gk_in[...] = jnp.zeros((2 * B * NGLOB, Dqk), jnp.bfloat16)

        barrier_sem = pltpu.get_barrier_semaphore()
        for nbr in (left, right):
            pl.semaphore_signal(
                barrier_sem, inc=1,
                device_id=(nbr,), device_id_type=pl.DeviceIdType.MESH,
            )
        pl.semaphore_wait(barrier_sem, 2)

        kown = k_ref[...].astype(jnp.bfloat16)
        vown = v_ref[...].astype(jnp.bfloat16)

        for b in range(B):
            ho_r[b * BAND:(b + 1) * BAND] = (
                kown[b * Skv + Skv - BAND:(b + 1) * Skv])
            ho_r[(B + b) * BAND:(B + b + 1) * BAND] = (
                vown[b * Skv + Skv - BAND:(b + 1) * Skv])
            ho_l[b * BAND:(b + 1) * BAND] = (
                kown[b * Skv:b * Skv + BAND])
            ho_l[(B + b) * BAND:(B + b + 1) * BAND] = (
                vown[b * Skv:b * Skv + BAND])

        def copy(src, dst, ssem, rsem, target):
            return pltpu.make_async_remote_copy(
                src_ref=src, dst_ref=dst, send_sem=ssem, recv_sem=rsem,
                device_id=(target,), device_id_type=pl.DeviceIdType.MESH,
            )

        h_r = copy(ho_r, hi_l, hsend.at[0], hrecv.at[0], right)
        h_l = copy(ho_l, hi_r, hsend.at[1], hrecv.at[1], left)
        h_r.start()
        h_l.start()

        for b in range(B):
            gk_out[b * NGLOB:(b + 1) * NGLOB] = (
                kown[b * Skv:b * Skv + NGLOB])
            gk_out[(B + b) * NGLOB:(B + b + 1) * NGLOB] = (
                vown[b * Skv:b * Skv + NGLOB])

        q = lax.dot_general(
            x_ref[...].astype(jnp.bfloat16),
            wq_ref[...].astype(jnp.bfloat16),
            (((1,), (0,)), ((), ())),
            preferred_element_type=jnp.float32,
        )
        q = (q * 0.125).astype(jnp.bfloat16)

        qg_local = jnp.concatenate(
            [q[0:NGLOB], q[Sq:Sq + NGLOB]], axis=0)
        qg_out[...] = qg_local

        qg_d = [copy(qg_out, qg_in, qgsend.at[j], qgrecv.at[0], t)
                for j, t in enumerate((1, 2, 3))]
        gk_d = [copy(gk_out, gk_in, gksend.at[j], gkrecv.at[0], t)
                for j, t in enumerate((1, 2, 3))]

        @pl.when(is_root)
        def _():
            for d in qg_d + gk_d:
                d.start()

        not_root = jnp.logical_not(is_root)
        iq = lax.broadcasted_iota(jnp.int32, (Sq, Skv), 0)
        ik = lax.broadcasted_iota(jnp.int32, (Sq, Skv), 1)
        qi = my * Sq + iq
        kj = my * Skv + ik
        m_own = (jnp.abs(qi - kj) <= BAND) | (kj < NGLOB)
        iqh = lax.broadcasted_iota(jnp.int32, (Sq, BAND), 0)
        ikh = lax.broadcasted_iota(jnp.int32, (Sq, BAND), 1)
        qi_h = my * Sq + iqh
        m_l = jnp.abs(qi_h - (left * Skv + Skv - BAND + ikh)) <= BAND
        m_r = jnp.abs(qi_h - (right * Skv + ikh)) <= BAND
        m_gk = jnp.full((Sq, NGLOB), True) & not_root
        top = jnp.concatenate([m_own, m_l, m_r, m_gk], axis=1)
        bot = jnp.concatenate(
            [jnp.full((NGLOB, Skv), True),
             jnp.full((NGLOB, NCOL - Skv), False)], axis=1)
        mask = jnp.concatenate([top, bot], axis=0)

        @pl.when(not_root)
        def _():
            qg_d[0].wait_recv()
            gk_d[0].wait_recv()

        h_r.wait_recv()
        h_l.wait_recv()

        qg_val = jnp.where(is_root, qg_local, qg_in[...])

        ls, accs = {}, {}
        for b in range(B):
            for h in range(Hq):
                hs = slice(h * Dh, (h + 1) * Dh)
                qcat = jnp.concatenate(
                    [q[b * Sq:(b + 1) * Sq, hs],
                     qg_val[b * NGLOB:(b + 1) * NGLOB, hs]], axis=0)
                kcat = jnp.concatenate(
                    [kown[b * Skv:(b + 1) * Skv, hs],
                     hi_l[b * BAND:(b + 1) * BAND, hs],
                     hi_r[b * BAND:(b + 1) * BAND, hs],
                     gk_in[b * NGLOB:(b + 1) * NGLOB, hs]], axis=0)
                vcat = jnp.concatenate(
                    [vown[b * Skv:(b + 1) * Skv, hs],
                     hi_l[(B + b) * BAND:(B + b + 1) * BAND, hs],
                     hi_r[(B + b) * BAND:(B + b + 1) * BAND, hs],
                     gk_in[(B + b) * NGLOB:(B + b + 1) * NGLOB, hs]],
                    axis=0)
                sc = lax.dot_general(
                    qcat, kcat, (((1,), (1,)), ((), ())),
                    preferred_element_type=jnp.float32,
                )
                w = jnp.exp(jnp.where(mask, sc, NEG_INF))
                l = jnp.sum(w, axis=1, keepdims=True)
                acc = lax.dot_general(
                    w.astype(jnp.bfloat16), vcat, (((1,), (0,)), ((), ())),
                    preferred_element_type=jnp.float32,
                )
                ls[b, h] = l
                accs[b, h] = acc
                r0 = (b * Hq + h) * NGLOB
                part_out[r0:r0 + NGLOB, 0:Dh] = (
                    acc[Sq:NROW].astype(jnp.bfloat16))
                part_out[r0:r0 + NGLOB, Dh:2 * Dh] = jnp.broadcast_to(
                    l[Sq:NROW], (NGLOB, Dh)).astype(jnp.bfloat16)

        pd = [copy(part_out, part_in.at[s], psend.at[0], precv.at[s], 0)
              for s in range(3)]
        for s in range(3):
            @pl.when(my == s + 1)
            def _(s=s):
                pd[s].start()

        @pl.when(is_root)
        def _():
            for s in range(3):
                pd[s].wait_recv()

        total = part_out[...].astype(jnp.float32)
        for s in range(3):
            total = total + part_in[s].astype(jnp.float32)

        wo_b = wo_ref[...].astype(jnp.bfloat16)
        for b in range(B):
            cols = []
            for h in range(Hq):
                l, acc = ls[b, h], accs[b, h]
                r0 = (b * Hq + h) * NGLOB
                gacc = total[r0:r0 + NGLOB, 0:Dh]
                gl = total[r0:r0 + NGLOB, Dh:Dh + 1]
                topc = jnp.where(is_root, gacc / gl,
                                 acc[0:NGLOB] / l[0:NGLOB])
                rest = acc[NGLOB:Sq] / l[NGLOB:Sq]
                cols.append(jnp.concatenate(
                    [topc.astype(jnp.bfloat16), rest.astype(jnp.bfloat16)],
                    axis=0))
            ctx_b = jnp.concatenate(cols, axis=1)
            out_ref[b * Sq:(b + 1) * Sq, :] = lax.dot_general(
                ctx_b, wo_b, (((1,), (0,)), ((), ())),
                preferred_element_type=jnp.float32,
            )

        h_r.wait_send()
        h_l.wait_send()

        @pl.when(is_root)
        def _():
            for d in qg_d + gk_d:
                d.wait_send()

        for s in range(3):
            @pl.when(my == s + 1)
            def _(s=s):
                pd[s].wait_send()

        @functools.partial(pl.run_scoped, sem2=pltpu.SemaphoreType.REGULAR)
        def _(sem2):
            for nbr in (left, right):
                pl.semaphore_signal(
                    sem2, inc=1,
                    device_id=(nbr,), device_id_type=pl.DeviceIdType.MESH,
                )
            pl.semaphore_wait(sem2, 2)

    out2 = pl.pallas_call(
        body,
        out_shape=jax.ShapeDtypeStruct((B * Sq, Dm), jnp.float32),
        in_specs=[pl.BlockSpec(memory_space=pltpu.VMEM)] * 5,
        out_specs=pl.BlockSpec(memory_space=pltpu.VMEM),
        scratch_shapes=[
            pltpu.VMEM((2 * B * BAND, Dqk), jnp.bfloat16),
            pltpu.VMEM((2 * B * BAND, Dqk), jnp.bfloat16),
            pltpu.VMEM((2 * B * BAND, Dqk), jnp.bfloat16),
            pltpu.VMEM((2 * B * BAND, Dqk), jnp.bfloat16),
            pltpu.VMEM((2 * B * NGLOB, Dqk), jnp.bfloat16),
            pltpu.VMEM((2 * B * NGLOB, Dqk), jnp.bfloat16),
            pltpu.VMEM((B * NGLOB, Dqk), jnp.bfloat16),
            pltpu.VMEM((B * NGLOB, Dqk), jnp.bfloat16),
            pltpu.VMEM((B * Hq * NGLOB, 2 * Dh), jnp.bfloat16),
            pltpu.VMEM((3, B * Hq * NGLOB, 2 * Dh), jnp.bfloat16),
            pltpu.SemaphoreType.DMA((2,)),
            pltpu.SemaphoreType.DMA((2,)),
            pltpu.SemaphoreType.DMA((3,)),
            pltpu.SemaphoreType.DMA((1,)),
            pltpu.SemaphoreType.DMA((3,)),
            pltpu.SemaphoreType.DMA((1,)),
            pltpu.SemaphoreType.DMA((1,)),
            pltpu.SemaphoreType.DMA((3,)),
        ],
        compiler_params=pltpu.CompilerParams(
            collective_id=0, vmem_limit_bytes=100 * 1024 * 1024,
        ),
    )(x2, Wq, k2, v2, Wo)

    return out2.reshape(B, Sq, Dm)
